# baseline (speedup 1.0000x reference)
import functools

import jax
import jax.numpy as jnp
from jax import lax
from jax.experimental import pallas as pl
from jax.experimental.pallas import tpu as pltpu
from jax.experimental.pallas import tpu_sc as plsc

N = 10000
E = 320000
D = 128
NW = 32
CH = 128
NCH = 80
NCH2 = NCH // 2
EPT = NCH * CH
EPAD = NW * EPT
NP = 10240
RPT = NP // 16

_f32 = jnp.float32
_i32 = jnp.int32

_MESH = plsc.VectorSubcoreMesh(core_axis_name="c", subcore_axis_name="s")


@functools.partial(
    pl.kernel,
    mesh=_MESH,
    out_type=jax.ShapeDtypeStruct((2, NP), _f32),
    scratch_types=[
        pltpu.VMEM((NCH, CH), _i32),
        pltpu.VMEM((CH,), _f32),
        pltpu.VMEM((RPT,), _f32),
        pltpu.VMEM_SHARED((NP,), _f32),
    ],
)
def _deg_call(dst_hbm, out_hbm, idx_v, ones_v, zero_v, deg_sh):
    c = lax.axis_index("c")
    s = lax.axis_index("s")
    wid = s * 2 + c

    pltpu.sync_copy(dst_hbm.at[wid], idx_v)

    def _fill_ones(i, _):
        ones_v[pl.ds(i * 16, 16)] = jnp.ones((16,), _f32)
        return _

    lax.fori_loop(0, CH // 16, _fill_ones, None)

    def _fill_zero(i, _):
        zero_v[pl.ds(i * 16, 16)] = jnp.zeros((16,), _f32)
        return _

    lax.fori_loop(0, RPT // 16, _fill_zero, None)

    pltpu.sync_copy(zero_v, deg_sh.at[pl.ds(s * RPT, RPT)])
    plsc.subcore_barrier()

    def _scatter(j, _):
        pltpu.sync_copy(ones_v, deg_sh.at[idx_v.at[j]], add=True)
        return _

    lax.fori_loop(0, NCH, _scatter, None)
    plsc.subcore_barrier()

    pltpu.sync_copy(deg_sh.at[pl.ds(s * RPT, RPT)],
                    out_hbm.at[c, pl.ds(s * RPT, RPT)])


@functools.partial(
    pl.kernel,
    mesh=_MESH,
    out_type=jax.ShapeDtypeStruct((2, NP, D), _f32),
    scratch_types=[
        pltpu.VMEM((NCH2, CH), _i32),
        pltpu.VMEM((NCH2, CH), _i32),
        pltpu.VMEM((CH, D), _f32),
        pltpu.VMEM((CH, D), _f32),
        pltpu.VMEM_SHARED((NP, D), _f32),
        pltpu.SemaphoreType.DMA,
        pltpu.SemaphoreType.DMA,
    ],
)
def _agg_call(src_hbm, dst_hbm, hp_hbm, out_hbm,
              src_v, dst_v, rows_a, rows_b, acc_sh, sem_a, sem_b):
    c = lax.axis_index("c")
    s = lax.axis_index("s")
    wid = s * 2 + c

    def _zrow(i, _):
        def _zcol(k, __):
            rows_a[i, pl.ds(k * 16, 16)] = jnp.zeros((16,), _f32)
            return __
        lax.fori_loop(0, D // 16, _zcol, None)
        return _

    lax.fori_loop(0, CH, _zrow, None)

    def _zinit(m, _):
        pltpu.sync_copy(rows_a, acc_sh.at[pl.ds(s * RPT + m * CH, CH)])
        return _

    lax.fori_loop(0, RPT // CH, _zinit, None)
    plsc.subcore_barrier()

    def _half(half, _):
        pltpu.sync_copy(src_hbm.at[wid, pl.ds(half * NCH2, NCH2)], src_v)
        pltpu.sync_copy(dst_hbm.at[wid, pl.ds(half * NCH2, NCH2)], dst_v)

        def _fire(j, buf, sem):
            h0 = pltpu.async_copy(
                hp_hbm.at[src_v.at[j, pl.ds(0, 64)]],
                buf.at[pl.ds(0, 64)], sem)
            h1 = pltpu.async_copy(
                hp_hbm.at[src_v.at[j, pl.ds(64, 64)]],
                buf.at[pl.ds(64, 64)], sem)
            return h0, h1

        _fire(0, rows_a, sem_a)

        def _wait(j, buf, sem):
            pltpu.make_async_copy(
                hp_hbm.at[src_v.at[j, pl.ds(0, 64)]],
                buf.at[pl.ds(0, 64)], sem).wait()
            pltpu.make_async_copy(
                hp_hbm.at[src_v.at[j, pl.ds(64, 64)]],
                buf.at[pl.ds(64, 64)], sem).wait()

        def _step(h, __):
            j = h * 2
            _fire(j + 1, rows_b, sem_b)
            _wait(j, rows_a, sem_a)
            pltpu.sync_copy(rows_a, acc_sh.at[dst_v.at[j]], add=True)

            @pl.when(j + 2 < NCH2)
            def _():
                _fire(j + 2, rows_a, sem_a)

            _wait(j + 1, rows_b, sem_b)
            pltpu.sync_copy(rows_b, acc_sh.at[dst_v.at[j + 1]], add=True)
            return __

        lax.fori_loop(0, NCH2 // 2, _step, None)
        return _

    lax.fori_loop(0, 2, _half, None)
    plsc.subcore_barrier()

    def _wb(m, _):
        r0 = s * RPT + m * CH
        pltpu.sync_copy(acc_sh.at[pl.ds(r0, CH)],
                        out_hbm.at[c, pl.ds(r0, CH)])
        return _

    lax.fori_loop(0, RPT // CH, _wb, None)


_BLK = 1000


def _mm_body(x_ref, w_ref, deg_ref, hp_ref, dis_ref):
    deg = deg_ref[0] + deg_ref[1] + 1.0
    dis = lax.rsqrt(deg)
    h = jnp.dot(x_ref[...], w_ref[...], preferred_element_type=_f32)
    hp_ref[...] = h * dis
    dis_ref[...] = dis


def _mm_call(x, w, deg3):
    return pl.pallas_call(
        _mm_body,
        grid=(N // _BLK,),
        in_specs=[
            pl.BlockSpec((_BLK, D), lambda r: (r, 0)),
            pl.BlockSpec((D, D), lambda r: (0, 0)),
            pl.BlockSpec((2, _BLK, 1), lambda r: (0, r, 0)),
        ],
        out_specs=[
            pl.BlockSpec((_BLK, D), lambda r: (r, 0)),
            pl.BlockSpec((_BLK, 1), lambda r: (r, 0)),
        ],
        out_shape=[
            jax.ShapeDtypeStruct((N, D), _f32),
            jax.ShapeDtypeStruct((N, 1), _f32),
        ],
    )(x, w, deg3)


def _mid_body(acc_ref, hp_ref, dis_ref, b_ref, w_ref, out_ref):
    agg = acc_ref[0] + acc_ref[1] + hp_ref[...]
    h = jnp.maximum(agg * dis_ref[...] + b_ref[...], 0.0)
    out_ref[...] = jnp.dot(h, w_ref[...], preferred_element_type=_f32) \
        * dis_ref[...]


def _mid_call(acc, hp, dis, b, w):
    return pl.pallas_call(
        _mid_body,
        grid=(N // _BLK,),
        in_specs=[
            pl.BlockSpec((2, _BLK, D), lambda r: (0, r, 0)),
            pl.BlockSpec((_BLK, D), lambda r: (r, 0)),
            pl.BlockSpec((_BLK, 1), lambda r: (r, 0)),
            pl.BlockSpec((1, D), lambda r: (0, 0)),
            pl.BlockSpec((D, D), lambda r: (0, 0)),
        ],
        out_specs=pl.BlockSpec((_BLK, D), lambda r: (r, 0)),
        out_shape=jax.ShapeDtypeStruct((N, D), _f32),
    )(acc, hp, dis, b, w)


def _out_body(acc_ref, hp_ref, dis_ref, b_ref, out_ref):
    agg = acc_ref[0] + acc_ref[1] + hp_ref[...]
    out_ref[...] = agg * dis_ref[...] + b_ref[...]


def _out_call(acc, hp, dis, b):
    return pl.pallas_call(
        _out_body,
        grid=(N // _BLK,),
        in_specs=[
            pl.BlockSpec((2, _BLK, D), lambda r: (0, r, 0)),
            pl.BlockSpec((_BLK, D), lambda r: (r, 0)),
            pl.BlockSpec((_BLK, 1), lambda r: (r, 0)),
            pl.BlockSpec((1, D), lambda r: (0, 0)),
        ],
        out_specs=pl.BlockSpec((_BLK, D), lambda r: (r, 0)),
        out_shape=jax.ShapeDtypeStruct((N, D), _f32),
    )(acc, hp, dis, b)


def _pad_rows(h):
    return jnp.concatenate([h, jnp.zeros((NP - N, D), _f32)], axis=0)


@jax.jit
def kernel(x, edge_index, W1, b1, W2, b2):
    src = edge_index[0].astype(_i32)
    dst = edge_index[1].astype(_i32)
    pad = jnp.full((EPAD - E,), N, _i32)
    src_r = jnp.concatenate([src, pad]).reshape(NW, NCH, CH)
    dst_r = jnp.concatenate([dst, pad]).reshape(NW, NCH, CH)

    degp = _deg_call(dst_r)
    deg3 = degp[:, :N].reshape(2, N, 1)

    hp1, dis = _mm_call(x, W1, deg3)
    acc1 = _agg_call(src_r, dst_r, _pad_rows(hp1))
    hp2 = _mid_call(acc1[:, :N], hp1, dis, b1.reshape(1, D), W2)
    acc2 = _agg_call(src_r, dst_r, _pad_rows(hp2))
    return _out_call(acc2[:, :N], hp2, dis, b2.reshape(1, D))


# speedup vs baseline: 1.0543x; 1.0543x over previous
import functools

import jax
import jax.numpy as jnp
from jax import lax
from jax.experimental import pallas as pl
from jax.experimental.pallas import tpu as pltpu
from jax.experimental.pallas import tpu_sc as plsc

N = 10000
E = 320000
D = 128
NW = 32
CH = 128
NCH = 80
NCH2 = NCH // 2
EPT = NCH * CH
EPAD = NW * EPT
NP = 10240
RPT = NP // 16

_f32 = jnp.float32
_i32 = jnp.int32

_MESH = plsc.VectorSubcoreMesh(core_axis_name="c", subcore_axis_name="s")


@functools.partial(
    pl.kernel,
    mesh=_MESH,
    out_type=jax.ShapeDtypeStruct((2, NP), _f32),
    scratch_types=[
        pltpu.VMEM((NCH, CH), _i32),
        pltpu.VMEM((CH,), _f32),
        pltpu.VMEM((RPT,), _f32),
        pltpu.VMEM_SHARED((NP,), _f32),
    ],
)
def _deg_call(dst_hbm, out_hbm, idx_v, ones_v, zero_v, deg_sh):
    c = lax.axis_index("c")
    s = lax.axis_index("s")
    wid = s * 2 + c

    pltpu.sync_copy(dst_hbm.at[wid], idx_v)

    def _fill_ones(i, _):
        ones_v[pl.ds(i * 16, 16)] = jnp.ones((16,), _f32)
        return _

    lax.fori_loop(0, CH // 16, _fill_ones, None)

    def _fill_zero(i, _):
        zero_v[pl.ds(i * 16, 16)] = jnp.zeros((16,), _f32)
        return _

    lax.fori_loop(0, RPT // 16, _fill_zero, None)

    pltpu.sync_copy(zero_v, deg_sh.at[pl.ds(s * RPT, RPT)])
    plsc.subcore_barrier()

    def _scatter(j, _):
        pltpu.sync_copy(ones_v, deg_sh.at[idx_v.at[j]], add=True)
        return _

    lax.fori_loop(0, NCH, _scatter, None)
    plsc.subcore_barrier()

    pltpu.sync_copy(deg_sh.at[pl.ds(s * RPT, RPT)],
                    out_hbm.at[c, pl.ds(s * RPT, RPT)])


@functools.partial(
    pl.kernel,
    mesh=_MESH,
    out_type=jax.ShapeDtypeStruct((2, NP, D), _f32),
    scratch_types=[
        pltpu.VMEM((NCH2, CH), _i32),
        pltpu.VMEM((NCH2, CH), _i32),
        pltpu.VMEM((CH, D), _f32),
        pltpu.VMEM((CH, D), _f32),
        pltpu.VMEM_SHARED((NP, D), _f32),
        pltpu.SemaphoreType.DMA,
        pltpu.SemaphoreType.DMA,
    ],
)
def _agg_call(src_hbm, dst_hbm, hp_hbm, out_hbm,
              src_v, dst_v, rows_a, rows_b, acc_sh, sem_a, sem_b):
    c = lax.axis_index("c")
    s = lax.axis_index("s")
    wid = s * 2 + c

    def _zrow(i, _):
        def _zcol(k, __):
            rows_a[i, pl.ds(k * 16, 16)] = jnp.zeros((16,), _f32)
            return __
        lax.fori_loop(0, D // 16, _zcol, None)
        return _

    lax.fori_loop(0, CH, _zrow, None)

    def _zinit(m, _):
        pltpu.sync_copy(rows_a, acc_sh.at[pl.ds(s * RPT + m * CH, CH)])
        return _

    lax.fori_loop(0, RPT // CH, _zinit, None)
    plsc.subcore_barrier()

    def _half(half, _):
        pltpu.sync_copy(src_hbm.at[wid, pl.ds(half * NCH2, NCH2)], src_v)
        pltpu.sync_copy(dst_hbm.at[wid, pl.ds(half * NCH2, NCH2)], dst_v)

        pltpu.async_copy(hp_hbm.at[src_v.at[0]], rows_a, sem_a)

        def _step(h, __):
            j = h * 2
            cpb = pltpu.async_copy(hp_hbm.at[src_v.at[j + 1]], rows_b, sem_b)
            pltpu.make_async_copy(hp_hbm.at[src_v.at[j]], rows_a, sem_a).wait()
            pltpu.sync_copy(rows_a, acc_sh.at[dst_v.at[j]], add=True)

            @pl.when(j + 2 < NCH2)
            def _():
                pltpu.async_copy(hp_hbm.at[src_v.at[j + 2]], rows_a, sem_a)

            cpb.wait()
            pltpu.sync_copy(rows_b, acc_sh.at[dst_v.at[j + 1]], add=True)
            return __

        lax.fori_loop(0, NCH2 // 2, _step, None)
        return _

    lax.fori_loop(0, 2, _half, None)
    plsc.subcore_barrier()

    def _wb(m, _):
        r0 = s * RPT + m * CH
        pltpu.sync_copy(acc_sh.at[pl.ds(r0, CH)],
                        out_hbm.at[c, pl.ds(r0, CH)])
        return _

    lax.fori_loop(0, RPT // CH, _wb, None)


_BLK = 1000


def _mm_body(x_ref, w_ref, deg_ref, hp_ref, dis_ref):
    deg = deg_ref[0] + deg_ref[1] + 1.0
    dis = lax.rsqrt(deg)
    h = jnp.dot(x_ref[...], w_ref[...], preferred_element_type=_f32)
    hp_ref[...] = h * dis
    dis_ref[...] = dis


def _mm_call(x, w, deg3):
    return pl.pallas_call(
        _mm_body,
        grid=(N // _BLK,),
        in_specs=[
            pl.BlockSpec((_BLK, D), lambda r: (r, 0)),
            pl.BlockSpec((D, D), lambda r: (0, 0)),
            pl.BlockSpec((2, _BLK, 1), lambda r: (0, r, 0)),
        ],
        out_specs=[
            pl.BlockSpec((_BLK, D), lambda r: (r, 0)),
            pl.BlockSpec((_BLK, 1), lambda r: (r, 0)),
        ],
        out_shape=[
            jax.ShapeDtypeStruct((NP, D), _f32),
            jax.ShapeDtypeStruct((NP, 1), _f32),
        ],
    )(x, w, deg3)


def _mid_body(acc_ref, hp_ref, dis_ref, b_ref, w_ref, out_ref):
    agg = acc_ref[0] + acc_ref[1] + hp_ref[...]
    h = jnp.maximum(agg * dis_ref[...] + b_ref[...], 0.0)
    out_ref[...] = jnp.dot(h, w_ref[...], preferred_element_type=_f32) \
        * dis_ref[...]


def _mid_call(acc, hp, dis, b, w):
    return pl.pallas_call(
        _mid_body,
        grid=(N // _BLK,),
        in_specs=[
            pl.BlockSpec((2, _BLK, D), lambda r: (0, r, 0)),
            pl.BlockSpec((_BLK, D), lambda r: (r, 0)),
            pl.BlockSpec((_BLK, 1), lambda r: (r, 0)),
            pl.BlockSpec((1, D), lambda r: (0, 0)),
            pl.BlockSpec((D, D), lambda r: (0, 0)),
        ],
        out_specs=pl.BlockSpec((_BLK, D), lambda r: (r, 0)),
        out_shape=jax.ShapeDtypeStruct((NP, D), _f32),
    )(acc, hp, dis, b, w)


def _out_body(acc_ref, hp_ref, dis_ref, b_ref, out_ref):
    agg = acc_ref[0] + acc_ref[1] + hp_ref[...]
    out_ref[...] = agg * dis_ref[...] + b_ref[...]


def _out_call(acc, hp, dis, b):
    return pl.pallas_call(
        _out_body,
        grid=(N // _BLK,),
        in_specs=[
            pl.BlockSpec((2, _BLK, D), lambda r: (0, r, 0)),
            pl.BlockSpec((_BLK, D), lambda r: (r, 0)),
            pl.BlockSpec((_BLK, 1), lambda r: (r, 0)),
            pl.BlockSpec((1, D), lambda r: (0, 0)),
        ],
        out_specs=pl.BlockSpec((_BLK, D), lambda r: (r, 0)),
        out_shape=jax.ShapeDtypeStruct((N, D), _f32),
    )(acc, hp, dis, b)


@jax.jit
def kernel(x, edge_index, W1, b1, W2, b2):
    src = edge_index[0].astype(_i32)
    dst = edge_index[1].astype(_i32)
    pad = jnp.full((EPAD - E,), N, _i32)
    src_r = jnp.concatenate([src, pad]).reshape(NW, NCH, CH)
    dst_r = jnp.concatenate([dst, pad]).reshape(NW, NCH, CH)

    degp = _deg_call(dst_r)
    deg3 = degp.reshape(2, NP, 1)

    hp1, dis = _mm_call(x, W1, deg3)
    acc1 = _agg_call(src_r, dst_r, hp1)
    hp2 = _mid_call(acc1, hp1, dis, b1.reshape(1, D), W2)
    acc2 = _agg_call(src_r, dst_r, hp2)
    return _out_call(acc2, hp2, dis, b2.reshape(1, D))
